# Initial kernel scaffold; baseline (speedup 1.0000x reference)
import jax
import jax.numpy as jnp
from jax.experimental import pallas as pl
from jax.experimental.pallas import tpu as pltpu

D = 768
E = 64
K = 2
DFF = 128
T = 2048
NUM_ITERS = 2
MIN_ENT = 0.8
EC = 8


def _router_kernel(state_ref, wr_ref, br_ref, noise_ref,
                   logits_ref, combine_ref, usage_ref, lb_ref):
    state = state_ref[...]
    logits = jax.lax.dot_general(
        state, wr_ref[...], (((1,), (0,)), ((), ())),
        preferred_element_type=jnp.float32) + br_ref[...]
    m = jnp.max(logits, axis=-1, keepdims=True)
    ex = jnp.exp(logits - m)
    probs = ex / jnp.sum(ex, axis=-1, keepdims=True)
    entropy = jnp.mean(-jnp.sum(probs * jnp.log(probs), axis=-1))
    logits_ref[...] = jnp.where(entropy < MIN_ENT, logits + noise_ref[...],
                                logits)
    iota = jax.lax.broadcasted_iota(jnp.int32, (T, E), 1)
    w1 = jnp.max(probs, axis=-1, keepdims=True)
    i1 = jnp.min(jnp.where(probs == w1, iota, E), axis=-1, keepdims=True)
    oh1 = iota == i1
    probs2 = jnp.where(oh1, -1.0, probs)
    w2 = jnp.max(probs2, axis=-1, keepdims=True)
    i2 = jnp.min(jnp.where(probs2 == w2, iota, E), axis=-1, keepdims=True)
    oh2 = iota == i2
    s = w1 + w2
    combine = (jnp.where(oh1, w1, 0.0) + jnp.where(oh2, w2, 0.0)) / s
    combine_ref[...] = combine
    counts = jnp.sum(oh1.astype(jnp.float32) + oh2.astype(jnp.float32),
                     axis=0, keepdims=True)
    usage_ref[...] = counts / T
    P = jnp.mean(probs, axis=0, keepdims=True)
    f = counts / (T * K)
    lb_ref[...] = jnp.sum(f * P).reshape(1, 1) * E


def _ffn_kernel(state_ref, comb_ref, b2_ref, w1_ref, b1_ref, w2_ref, out_ref):
    g = pl.program_id(0)

    @pl.when(g == 0)
    def _init():
        out_ref[...] = state_ref[...] + jax.lax.dot_general(
            comb_ref[...], b2_ref[...], (((1,), (0,)), ((), ())),
            preferred_element_type=jnp.float32)

    h = jax.lax.dot_general(
        state_ref[...], w1_ref[...], (((1,), (0,)), ((), ())),
        preferred_element_type=jnp.float32) + b1_ref[...]
    a = jnp.maximum(h, 0.0)
    cchunk = comb_ref[:, pl.ds(g * EC, EC)]
    parts = []
    for j in range(EC):
        parts.append(a[:, j * DFF:(j + 1) * DFF] * cchunk[:, j:j + 1])
    a = jnp.concatenate(parts, axis=1)
    out_ref[...] += jax.lax.dot_general(
        a, w2_ref[...], (((1,), (0,)), ((), ())),
        preferred_element_type=jnp.float32)


def _router_call(state, Wr, br2, noise):
    return pl.pallas_call(
        _router_kernel,
        out_shape=[
            jax.ShapeDtypeStruct((T, E), jnp.float32),
            jax.ShapeDtypeStruct((T, E), jnp.float32),
            jax.ShapeDtypeStruct((1, E), jnp.float32),
            jax.ShapeDtypeStruct((1, 1), jnp.float32),
        ],
    )(state, Wr, br2, noise)


def _ffn_call(state, combine, b2, W1c, b1c, W2c):
    nsteps = E // EC
    return pl.pallas_call(
        _ffn_kernel,
        grid=(nsteps,),
        in_specs=[
            pl.BlockSpec((T, D), lambda g: (0, 0)),
            pl.BlockSpec((T, E), lambda g: (0, 0)),
            pl.BlockSpec((E, D), lambda g: (0, 0)),
            pl.BlockSpec((D, EC * DFF), lambda g: (0, g)),
            pl.BlockSpec((1, EC * DFF), lambda g: (0, g)),
            pl.BlockSpec((EC * DFF, D), lambda g: (g, 0)),
        ],
        out_specs=pl.BlockSpec((T, D), lambda g: (0, 0)),
        out_shape=jax.ShapeDtypeStruct((T, D), jnp.float32),
        compiler_params=pltpu.CompilerParams(
            dimension_semantics=("arbitrary",)),
    )(state, combine, b2, W1c, b1c, W2c)


def kernel(x, Wr, br, W1, b1, W2, b2):
    B, S, Dm = x.shape
    state = x.reshape(T, D)
    W1c = W1.transpose(1, 0, 2).reshape(D, E * DFF)
    W2c = W2.reshape(E * DFF, D)
    b1c = b1.reshape(1, E * DFF)
    br2 = br.reshape(1, E)
    all_logits, all_usage, all_states = [], [], []
    lb = None
    for it in range(NUM_ITERS):
        noise = jax.random.normal(
            jax.random.fold_in(jax.random.key(1), it), (T, E),
            dtype=jnp.float32) * 0.1
        logits, combine, usage, lb = _router_call(state, Wr, br2, noise)
        state = _ffn_call(state, combine, b2, W1c, b1c, W2c)
        all_logits.append(logits)
        all_usage.append(usage.reshape(E))
        all_states.append(state)
    final_output = state.reshape(B, S, Dm)
    return (final_output, lb.reshape(()), jnp.stack(all_logits),
            jnp.stack(all_usage), jnp.stack(all_states))


# speedup vs baseline: 5.8030x; 5.8030x over previous
import jax
import jax.numpy as jnp
from jax.experimental import pallas as pl
from jax.experimental.pallas import tpu as pltpu

D = 768
E = 64
K = 2
DFF = 128
T = 2048
NUM_ITERS = 2
MIN_ENT = 0.8
EC = 8


def _router_kernel(state_ref, wr_ref, br_ref, noise_ref,
                   logits_ref, combine_ref, usage_ref, lb_ref):
    state = state_ref[...]
    logits = jax.lax.dot_general(
        state, wr_ref[...], (((1,), (0,)), ((), ())),
        preferred_element_type=jnp.float32) + br_ref[...]
    m = jnp.max(logits, axis=-1, keepdims=True)
    ex = jnp.exp(logits - m)
    probs = ex / jnp.sum(ex, axis=-1, keepdims=True)
    entropy = jnp.mean(-jnp.sum(probs * jnp.log(probs), axis=-1))
    logits_ref[...] = jnp.where(entropy < MIN_ENT, logits + noise_ref[...],
                                logits)
    iota = jax.lax.broadcasted_iota(jnp.int32, (T, E), 1)
    w1 = jnp.max(probs, axis=-1, keepdims=True)
    i1 = jnp.min(jnp.where(probs == w1, iota, E), axis=-1, keepdims=True)
    oh1 = iota == i1
    probs2 = jnp.where(oh1, -1.0, probs)
    w2 = jnp.max(probs2, axis=-1, keepdims=True)
    i2 = jnp.min(jnp.where(probs2 == w2, iota, E), axis=-1, keepdims=True)
    oh2 = iota == i2
    s = w1 + w2
    combine = (jnp.where(oh1, w1, 0.0) + jnp.where(oh2, w2, 0.0)) / s
    combine_ref[...] = combine
    counts = jnp.sum(oh1.astype(jnp.float32) + oh2.astype(jnp.float32),
                     axis=0, keepdims=True)
    usage_ref[...] = counts / T
    P = jnp.mean(probs, axis=0, keepdims=True)
    f = counts / (T * K)
    lb_ref[...] = jnp.sum(f * P).reshape(1, 1) * E


def _ffn_kernel(state_ref, comb_ref, b2_ref, w1_ref, b1_ref, w2_ref, out_ref):
    g = pl.program_id(0)

    @pl.when(g == 0)
    def _init():
        out_ref[...] = state_ref[...] + jax.lax.dot_general(
            comb_ref[...], b2_ref[...], (((1,), (0,)), ((), ())),
            preferred_element_type=jnp.float32)

    h = jax.lax.dot_general(
        state_ref[...], w1_ref[...], (((1,), (0,)), ((), ())),
        preferred_element_type=jnp.float32) + b1_ref[...]
    a = jnp.maximum(h, 0.0)
    erow = jax.lax.broadcasted_iota(jnp.int32, (E, EC * DFF), 0)
    ecol = jax.lax.broadcasted_iota(jnp.int32, (E, EC * DFF), 1) // DFF
    expand = (erow == ecol + g * EC).astype(jnp.float32)
    scale = jax.lax.dot_general(
        comb_ref[...], expand, (((1,), (0,)), ((), ())),
        preferred_element_type=jnp.float32)
    out_ref[...] += jax.lax.dot_general(
        a * scale, w2_ref[...], (((1,), (0,)), ((), ())),
        preferred_element_type=jnp.float32)


def _router_call(state, Wr, br2, noise):
    return pl.pallas_call(
        _router_kernel,
        out_shape=[
            jax.ShapeDtypeStruct((T, E), jnp.float32),
            jax.ShapeDtypeStruct((T, E), jnp.float32),
            jax.ShapeDtypeStruct((1, E), jnp.float32),
            jax.ShapeDtypeStruct((1, 1), jnp.float32),
        ],
    )(state, Wr, br2, noise)


def _ffn_call(state, combine, b2, W1c, b1c, W2c):
    nsteps = E // EC
    return pl.pallas_call(
        _ffn_kernel,
        grid=(nsteps,),
        in_specs=[
            pl.BlockSpec((T, D), lambda g: (0, 0)),
            pl.BlockSpec((T, E), lambda g: (0, 0)),
            pl.BlockSpec((E, D), lambda g: (0, 0)),
            pl.BlockSpec((D, EC * DFF), lambda g: (0, g)),
            pl.BlockSpec((1, EC * DFF), lambda g: (0, g)),
            pl.BlockSpec((EC * DFF, D), lambda g: (g, 0)),
        ],
        out_specs=pl.BlockSpec((T, D), lambda g: (0, 0)),
        out_shape=jax.ShapeDtypeStruct((T, D), jnp.float32),
        compiler_params=pltpu.CompilerParams(
            dimension_semantics=("arbitrary",)),
    )(state, combine, b2, W1c, b1c, W2c)


def kernel(x, Wr, br, W1, b1, W2, b2):
    B, S, Dm = x.shape
    state = x.reshape(T, D)
    W1c = W1.transpose(1, 0, 2).reshape(D, E * DFF)
    W2c = W2.reshape(E * DFF, D)
    b1c = b1.reshape(1, E * DFF)
    br2 = br.reshape(1, E)
    all_logits, all_usage, all_states = [], [], []
    lb = None
    for it in range(NUM_ITERS):
        noise = jax.random.normal(
            jax.random.fold_in(jax.random.key(1), it), (T, E),
            dtype=jnp.float32) * 0.1
        logits, combine, usage, lb = _router_call(state, Wr, br2, noise)
        state = _ffn_call(state, combine, b2, W1c, b1c, W2c)
        all_logits.append(logits)
        all_usage.append(usage.reshape(E))
        all_states.append(state)
    final_output = state.reshape(B, S, Dm)
    return (final_output, lb.reshape(()), jnp.stack(all_logits),
            jnp.stack(all_usage), jnp.stack(all_states))
